# baseline (speedup 1.0000x reference)
import functools

import jax
import jax.numpy as jnp
from jax import lax
from jax.experimental import pallas as pl
from jax.experimental.pallas import tpu as pltpu, tpu_sc as plsc

NC = 2
NS = 16
L = 16
NT = NC * NS

B = 16384
D = 64
C = 100000
CP = 100352
ZPT = CP // NS
RPT = B // NT
LPT = B // NS
YROWS = B // 128


def _body(y128, batch_h, centers_h, out_h,
          hist_sh, acc_sh,
          yh, yi, ones, zbuf, ctr, bat, cnt, wbuf, tacc, sbuf,
          sem_ctr, sem_bat, sem_cnt):
    cid = lax.axis_index("c")
    sid = lax.axis_index("s")
    wid = sid * NC + cid

    @pl.loop(0, ZPT // L)
    def _zero(j):
        zbuf[pl.ds(j * L, L)] = jnp.zeros((L,), jnp.float32)

    for k in range(LPT // 128):
        for j in range(128 // L):
            ones[k, pl.ds(j * L, L)] = jnp.ones((L,), jnp.float32)

    pltpu.sync_copy(y128.at[pl.ds(sid * (LPT // 128), LPT // 128)], yh)
    pltpu.sync_copy(y128.at[pl.ds(wid * (RPT // 128), RPT // 128)], yi)
    pltpu.sync_copy(zbuf, hist_sh.at[pl.ds(sid * ZPT, ZPT)])

    plsc.subcore_barrier()

    for k in range(LPT // 128):
        pltpu.sync_copy(ones.at[k], hist_sh.at[yh.at[k]], add=True)

    plsc.subcore_barrier()

    cps = []
    for k in range(RPT // 128):
        cps.append(pltpu.async_copy(
            centers_h.at[yi.at[k]], ctr.at[pl.ds(k * 128, 128)], sem_ctr))
        cps.append(pltpu.async_copy(
            hist_sh.at[yi.at[k]], cnt.at[pl.ds(k * 128, 128)], sem_cnt))
    bp = pltpu.async_copy(batch_h.at[pl.ds(wid * RPT, RPT)], bat, sem_bat)
    for cp in cps:
        cp.wait()
    bp.wait()

    @pl.loop(0, RPT // L)
    def _w(j):
        c16 = cnt[pl.ds(j * L, L)]
        wbuf[pl.ds(j * L, L)] = 1.0 / (1.0 + c16)

    zero16 = jnp.zeros((L,), jnp.float32)

    def row(i, accs):
        a0, a1, a2, a3 = accs
        wv = plsc.load_gather(wbuf, [jnp.full((L,), i, jnp.int32)])
        d0 = bat[i, pl.ds(0 * L, L)] - ctr[i, pl.ds(0 * L, L)]
        d1 = bat[i, pl.ds(1 * L, L)] - ctr[i, pl.ds(1 * L, L)]
        d2 = bat[i, pl.ds(2 * L, L)] - ctr[i, pl.ds(2 * L, L)]
        d3 = bat[i, pl.ds(3 * L, L)] - ctr[i, pl.ds(3 * L, L)]
        return (a0 + wv * d0 * d0, a1 + wv * d1 * d1,
                a2 + wv * d2 * d2, a3 + wv * d3 * d3)

    a0, a1, a2, a3 = lax.fori_loop(
        0, RPT, row, (zero16, zero16, zero16, zero16))
    tacc[...] = (a0 + a1) + (a2 + a3)

    pltpu.sync_copy(tacc, acc_sh.at[sid])
    plsc.subcore_barrier()

    @pl.when(sid == 0)
    def _reduce():
        pltpu.sync_copy(acc_sh, sbuf)
        total = sbuf[0, :]
        for k in range(1, NS):
            total = total + sbuf[k, :]
        tacc[...] = total
        pltpu.sync_copy(tacc, out_h.at[cid])


@jax.jit
def _center_loss(y, batch, centers):
    y128 = y.reshape(YROWS, 128).astype(jnp.int32)
    mesh = plsc.VectorSubcoreMesh(
        core_axis_name="c", subcore_axis_name="s",
        num_cores=NC, num_subcores=NS)
    partials = pl.kernel(
        _body,
        out_type=jax.ShapeDtypeStruct((NC, L), jnp.float32),
        mesh=mesh,
        compiler_params=pltpu.CompilerParams(
            needs_layout_passes=False, use_tc_tiling_on_sc=False),
        scratch_types=[
            pltpu.VMEM_SHARED((CP,), jnp.float32),
            pltpu.VMEM_SHARED((NS, L), jnp.float32),
            pltpu.VMEM((LPT // 128, 128), jnp.int32),
            pltpu.VMEM((RPT // 128, 128), jnp.int32),
            pltpu.VMEM((LPT // 128, 128), jnp.float32),
            pltpu.VMEM((ZPT,), jnp.float32),
            pltpu.VMEM((RPT, D), jnp.float32),
            pltpu.VMEM((RPT, D), jnp.float32),
            pltpu.VMEM((RPT,), jnp.float32),
            pltpu.VMEM((RPT,), jnp.float32),
            pltpu.VMEM((L,), jnp.float32),
            pltpu.VMEM((NS, L), jnp.float32),
            pltpu.SemaphoreType.DMA,
            pltpu.SemaphoreType.DMA,
            pltpu.SemaphoreType.DMA,
        ],
    )(y128, batch, centers)
    return 0.5 * jnp.sum(partials)


def kernel(y, batch, centers):
    return _center_loss(y, batch, centers)
